# baseline (speedup 1.0000x reference)
import jax
import jax.numpy as jnp
from jax.experimental import pallas as pl
from jax.experimental.pallas import tpu as pltpu


IMGS = 2

_G44 = (44, 44, 48, 2216, 49, 2112)
_G22 = (22, 22, 24, 584, 25, 528)
_G11 = (11, 11, 13, 176, 14, 143)


def _mp(geom):
    return (IMGS - 1) * geom[3] + geom[5]


def _taps(src, base, Wp, MP, Cin, KH, KW, w_ref, b_ref, relu, im=None):
    def slab(kh, kw):
        return src[pl.ds(base + kh * Wp + kw, MP), pl.ds(0, Cin)]

    if im is not None:
        T = KH * KW
        for kh in range(KH):
            for kw in range(KW):
                im[pl.ds(0, MP), pl.ds((kh * KW + kw) * Cin, Cin)] = (
                    slab(kh, kw))
        res = jnp.dot(im[pl.ds(0, MP), pl.ds(0, T * Cin)], w_ref[...],
                      preferred_element_type=jnp.float32) + b_ref[...]
    else:
        res = None
        for kh in range(KH):
            for kw in range(KW):
                d = jnp.dot(slab(kh, kw), w_ref[kh, kw],
                            preferred_element_type=jnp.float32)
                res = (d + b_ref[...]) if res is None else (res + d)
    if relu:
        res = jnp.maximum(res, 0.0)
    return res


def _store(dst, val, geom, lane_off, C):
    PAD, off, M = geom[3], geom[4], geom[5]
    for i in range(IMGS):
        dst[pl.ds(i * PAD + off, M),
            pl.ds(lane_off, C)] = val[i * PAD:i * PAD + M].astype(
                jnp.bfloat16)


def _halo(dst, geom, C):
    H, W, Wp, PAD, off, M = geom
    side = jnp.zeros((Wp - W, C), jnp.bfloat16)
    for i in range(IMGS):
        b = i * PAD
        dst[pl.ds(b, Wp + 1), pl.ds(0, C)] = jnp.zeros((Wp + 1, C),
                                                       jnp.bfloat16)
        for r in range(1, H + 1):
            dst[pl.ds(b + r * Wp + W + 1, Wp - W), pl.ds(0, C)] = side
        t0 = (H + 1) * Wp + 1
        dst[pl.ds(b + t0, PAD - t0), pl.ds(0, C)] = jnp.zeros(
            (PAD - t0, C), jnp.bfloat16)


def _conv_to(src, sgeom, dst, dgeom, Cin, w_ref, b_ref, relu,
             lane_off=0, O=None, im=None):
    res = _taps(src, 0, sgeom[2], _mp(sgeom), Cin, 3, 3, w_ref, b_ref,
                relu, im=im)
    _store(dst, res, dgeom, lane_off, O)


def _max_pool_2x2(src, sgeom, dst, dgeom, C):
    H, W, Wp, sPAD = sgeom[0], sgeom[1], sgeom[2], sgeom[3]
    OH, OW = H // 2, W // 2
    ri = jax.lax.broadcasted_iota(jnp.int32, (OW, W), 0)
    cj = jax.lax.broadcasted_iota(jnp.int32, (OW, W), 1)
    sel_e = jnp.where(cj == 2 * ri, 1.0, 0.0).astype(jnp.bfloat16)
    sel_o = jnp.where(cj == 2 * ri + 1, 1.0, 0.0).astype(jnp.bfloat16)
    dWp, dPAD = dgeom[2], dgeom[3]
    for i in range(IMGS):
        for oh in range(OH):
            r0 = src[pl.ds(i * sPAD + (2 * oh + 1) * Wp + 1, W),
                     pl.ds(0, C)]
            r1 = src[pl.ds(i * sPAD + (2 * oh + 2) * Wp + 1, W),
                     pl.ds(0, C)]
            rm = jnp.maximum(r0, r1)
            e = jnp.dot(sel_e, rm, preferred_element_type=jnp.float32)
            o = jnp.dot(sel_o, rm, preferred_element_type=jnp.float32)
            dst[pl.ds(i * dPAD + (oh + 1) * dWp + 1, OW),
                pl.ds(0, C)] = jnp.maximum(e, o).astype(jnp.bfloat16)


def _max_pool_3x3(src, dst, geom, C):
    Wp, PAD, off, M = geom[2], geom[3], geom[4], geom[5]
    MP = _mp(geom)
    m = None
    for i in range(3):
        for j in range(3):
            sl = src[pl.ds(i * Wp + j, MP), pl.ds(0, C)]
            m = sl if m is None else jnp.maximum(m, sl)
    for i in range(IMGS):
        dst[pl.ds(i * PAD + off, M), pl.ds(0, C)] = m[i * PAD:i * PAD + M]


def _feat(src, sgeom, w_ref, b_ref, dst, fpc, Cin, im=None):
    H, W, Wp, PAD, off, M = sgeom
    MP = _mp(sgeom)
    f = _taps(src, 0, Wp, MP, Cin, 3, 3, w_ref, b_ref, True, im=im)
    _store(dst, f, sgeom, 0, 64)
    fpc[pl.ds(0, IMGS * PAD), pl.ds(0, 64)] = jnp.zeros(
        (IMGS * PAD, 64), jnp.float32)
    for i in range(IMGS):
        for oh in range(H):
            fpc[pl.ds(i * PAD + (oh + 1) * Wp + 1, W), pl.ds(0, 64)] = (
                f[i * PAD + oh * Wp:i * PAD + oh * Wp + W])
    s = None
    for i in range(3):
        for j in range(3):
            sl = fpc[pl.ds(i * Wp + j, MP), pl.ds(0, 64)]
            s = sl if s is None else s + sl
    _store(dst, f - s * (1.0 / 9.0), sgeom, 64, 64)


def kernel(x, base0_w, base0_b, base1_w, base1_b, base2_w, base2_b,
           base3_w, base3_b, base4_w, base4_b, base5_w, base5_b,
           base6_w, base6_b, base7_w, base7_b, base8_w, base8_b,
           base9_w, base9_b, base10_w, base10_b, base11_w, base11_b,
           base12_w, base12_b,
           feat0_w, feat0_b, feat1_w, feat1_b, feat2_w, feat2_b,
           feat3_w, feat3_b, feat4_w, feat4_b,
           pool0_w, pool0_b, pool1_w, pool1_b, pool2_w, pool2_b,
           pool3_w, pool3_b, pool4_w, pool4_b,
           glob0_w, glob0_b, glob1_w, glob1_b, glob2_w, glob2_b,
           conv_g_w, conv_g_b, conv_l_w, conv_l_b):
    N = x.shape[0]
    G = N // IMGS
    x_nhwc = jnp.transpose(x.astype(jnp.float32), (0, 2, 3, 1))
    x_pad = jnp.pad(x_nhwc, ((0, 0), (1, 1), (1, 3), (0, 0)))
    x_pad = x_pad.reshape(N, 46 * 48, 3)
    x_pad = jnp.pad(x_pad, ((0, 0), (0, 106), (0, 0))).astype(jnp.bfloat16)
    cols = []
    for kh in range(3):
        for kw in range(3):
            d = kh * 48 + kw
            cols.append(jax.lax.dynamic_slice_in_dim(x_pad, d, 2216, axis=1))
    x_im = jnp.concatenate(cols, axis=-1)
    x_im = x_im.reshape(G, IMGS * 2216, 27)

    base_w = [base0_w, base1_w, base2_w, base3_w, base4_w, base5_w, base6_w,
              base7_w, base8_w, base9_w, base10_w, base11_w, base12_w]
    base_b = [base0_b, base1_b, base2_b, base3_b, base4_b, base5_b, base6_b,
              base7_b, base8_b, base9_b, base10_b, base11_b, base12_b]
    feat_w = [feat0_w, feat1_w, feat2_w, feat3_w, feat4_w]
    feat_b = [feat0_b, feat1_b, feat2_b, feat3_b, feat4_b]
    pool_w = [pool0_w, pool1_w, pool2_w, pool3_w, pool4_w]
    pool_b = [pool0_b, pool1_b, pool2_b, pool3_b, pool4_b]
    glob_w = [glob0_w, glob1_w, glob2_w]
    glob_b = [glob0_b, glob1_b, glob2_b]

    operands = [x_im]
    for w, b in zip(base_w + feat_w + pool_w + glob_w,
                    base_b + feat_b + pool_b + glob_b):
        KH, KW, Cin, O = w.shape
        wb = w.astype(jnp.bfloat16)
        if Cin <= 128:
            wb = wb.reshape(KH * KW * Cin, O)
        operands.append(wb)
        operands.append(b.reshape(1, -1).astype(jnp.float32))
    operands.append(conv_l_w.reshape(1, 640).astype(jnp.float32))
    operands.append(conv_g_w.reshape(1, 128).astype(jnp.float32))
    operands.append((conv_l_b + conv_g_b).reshape(1, 1).astype(jnp.float32))

    in_specs = [pl.BlockSpec((1, IMGS * 2216, 27), lambda n: (n, 0, 0))]
    for op in operands[1:]:
        in_specs.append(
            pl.BlockSpec(op.shape, lambda n, nd=op.ndim: (0,) * nd))

    def body(*refs):
        x_ref = refs[0]
        wr = refs[1:53]
        wl_ref, wg_ref, sb_ref = refs[53], refs[54], refs[55]
        prob_ref = refs[56]
        (c44a, c44b, c22a, c22b, c11a, c11b,
         dc0, dc1, dc2, dc3, dc4, gb1, gb2, fpc,
         im44, imm, img) = refs[57:]

        bw = [(wr[2 * i], wr[2 * i + 1]) for i in range(13)]
        fw = [(wr[2 * i + 26], wr[2 * i + 27]) for i in range(5)]
        pw = [(wr[2 * i + 36], wr[2 * i + 37]) for i in range(5)]
        gw = [(wr[2 * i + 46], wr[2 * i + 47]) for i in range(3)]

        res0 = jnp.dot(x_ref[0, pl.ds(0, _mp(_G44)), :], bw[0][0][...],
                       preferred_element_type=jnp.float32) + bw[0][1][...]
        _store(c44a, jnp.maximum(res0, 0.0), _G44, 0, 16)
        _halo(c44a, _G44, 16)
        _conv_to(c44a, _G44, c44b, _G44, 16, *bw[1], True, O=16, im=im44)
        _halo(c44b, _G44, 16)
        _max_pool_2x2(c44b, _G44, c22a, _G22, 16)
        _halo(c22a, _G22, 16)
        _feat(c22a, _G22, *fw[0], dc0, fpc, 16, im=imm)
        _conv_to(c22a, _G22, c22b, _G22, 16, *bw[2], True, O=32, im=imm)
        _halo(c22b, _G22, 32)
        _conv_to(c22b, _G22, c22a, _G22, 32, *bw[3], True, O=32, im=imm)
        _halo(c22a, _G22, 32)
        _max_pool_2x2(c22a, _G22, c11a, _G11, 32)
        _halo(c11a, _G11, 32)
        _feat(c11a, _G11, *fw[1], dc1, fpc, 32, im=imm)
        _conv_to(c11a, _G11, c11b, _G11, 32, *bw[4], True, O=32, im=imm)
        _halo(c11b, _G11, 32)
        _conv_to(c11b, _G11, c11a, _G11, 32, *bw[5], True, O=32, im=imm)
        _halo(c11a, _G11, 32)
        _conv_to(c11a, _G11, c11b, _G11, 32, *bw[6], True, O=32, im=imm)
        _halo(c11b, _G11, 32)
        _max_pool_3x3(c11b, c11a, _G11, 32)
        _halo(c11a, _G11, 32)
        _feat(c11a, _G11, *fw[2], dc2, fpc, 32, im=imm)
        _conv_to(c11a, _G11, c11b, _G11, 32, *bw[7], True, O=64, im=imm)
        _halo(c11b, _G11, 64)
        _conv_to(c11b, _G11, c11a, _G11, 64, *bw[8], True, O=64, im=imm)
        _halo(c11a, _G11, 64)
        _conv_to(c11a, _G11, c11b, _G11, 64, *bw[9], True, O=64, im=imm)
        _halo(c11b, _G11, 64)
        _max_pool_3x3(c11b, c11a, _G11, 64)
        _halo(c11a, _G11, 64)
        _feat(c11a, _G11, *fw[3], dc3, fpc, 64, im=imm)
        _conv_to(c11a, _G11, c11b, _G11, 64, *bw[10], True, O=512, im=imm)
        _halo(c11b, _G11, 512)
        _conv_to(c11b, _G11, c11a, _G11, 512, *bw[11], True, O=512)
        _halo(c11a, _G11, 512)
        _conv_to(c11a, _G11, c11b, _G11, 512, *bw[12], True, O=512)
        _halo(c11b, _G11, 512)
        _max_pool_3x3(c11b, c11a, _G11, 512)
        _halo(c11a, _G11, 512)
        _feat(c11a, _G11, *fw[4], dc4, fpc, 512)

        g0 = _taps(c11a, 14, 13, (IMGS - 1) * 176 + 85, 512, 5, 5,
                   *gw[0], True)
        for i in range(IMGS):
            gb1[pl.ds(i * 88, 85), :] = g0[i * 176:i * 176 + 85].astype(
                jnp.bfloat16)
        g1 = _taps(gb1, 0, 13, (IMGS - 1) * 88 + 29, 128, 5, 5,
                   *gw[1], True, im=img)
        for i in range(IMGS):
            gb2[pl.ds(i * 32, 29), :] = g1[i * 88:i * 88 + 29].astype(
                jnp.bfloat16)
        g2 = _taps(gb2, 0, 13, (IMGS - 1) * 32 + 1, 128, 3, 3,
                   *gw[2], False, im=imm)
        gsv = jnp.sum(g2 * wg_ref[...], axis=-1, keepdims=True)

        _halo(dc4, _G11, 128)
        d = _taps(dc4, 0, 13, _mp(_G11), 128, 3, 3, *pw[4], True, im=imm)
        for k, dck in ((3, dc3), (2, dc2), (1, dc1)):
            _store(dck, d, _G11, 128, 128)
            _halo(dck, _G11, 256)
            d = _taps(dck, 0, 13, _mp(_G11), 256, 3, 3, *pw[k], True)
        ri = jax.lax.broadcasted_iota(jnp.int32, (22, 11), 0)
        cj = jax.lax.broadcasted_iota(jnp.int32, (22, 11), 1)
        sel = jnp.where(ri // 2 == cj, 1.0, 0.0)
        for i in range(IMGS):
            for hs in range(11):
                b0 = i * 176 + hs * 13
                row = jnp.dot(sel, d[b0:b0 + 11],
                              preferred_element_type=jnp.float32)
                row = row.astype(jnp.bfloat16)
                for r in range(2):
                    dc0[pl.ds(i * 584 + (2 * hs + r + 1) * 24 + 1, 22),
                        pl.ds(128, 128)] = row
        _halo(dc0, _G22, 256)
        out = _taps(dc0, 0, 24, _mp(_G22), 256, 3, 3, *pw[0], False)

        s = jnp.sum(out * wl_ref[...], axis=-1, keepdims=True)
        for i in range(IMGS):
            z = (s[i * 584:i * 584 + 528] + gsv[i * 32:i * 32 + 1]
                 + sb_ref[...])
            prob_ref[i] = 1.0 / (1.0 + jnp.exp(-z))

    scratch = [
        pltpu.VMEM((IMGS * 2216, 16), jnp.bfloat16),
        pltpu.VMEM((IMGS * 2216, 16), jnp.bfloat16),
        pltpu.VMEM((IMGS * 584, 32), jnp.bfloat16),
        pltpu.VMEM((IMGS * 584, 32), jnp.bfloat16),
        pltpu.VMEM((IMGS * 176, 512), jnp.bfloat16),
        pltpu.VMEM((IMGS * 176, 512), jnp.bfloat16),
        pltpu.VMEM((IMGS * 584, 256), jnp.bfloat16),
        pltpu.VMEM((IMGS * 176, 256), jnp.bfloat16),
        pltpu.VMEM((IMGS * 176, 256), jnp.bfloat16),
        pltpu.VMEM((IMGS * 176, 256), jnp.bfloat16),
        pltpu.VMEM((IMGS * 176, 256), jnp.bfloat16),
        pltpu.VMEM((IMGS * 88, 128), jnp.bfloat16),
        pltpu.VMEM((IMGS * 32, 128), jnp.bfloat16),
        pltpu.VMEM((IMGS * 584, 64), jnp.float32),
        pltpu.VMEM(((IMGS - 1) * 2216 + 2112, 144), jnp.bfloat16),
        pltpu.VMEM(((IMGS - 1) * 584 + 528, 1152), jnp.bfloat16),
        pltpu.VMEM(((IMGS - 1) * 88 + 32, 3200), jnp.bfloat16),
    ]

    prob = pl.pallas_call(
        body,
        out_shape=jax.ShapeDtypeStruct((N, 528, 1), jnp.float32),
        grid=(G,),
        in_specs=in_specs,
        out_specs=pl.BlockSpec((IMGS, 528, 1), lambda n: (n, 0, 0)),
        scratch_shapes=scratch,
        compiler_params=pltpu.CompilerParams(
            dimension_semantics=("parallel",),
            vmem_limit_bytes=100 * 1024 * 1024),
    )(*operands)

    prob = prob.reshape(N, 22, 24)[:, :, :22]
    return prob[:, None, :, :]


# speedup vs baseline: 1.1566x; 1.1566x over previous
import jax
import jax.numpy as jnp
from jax.experimental import pallas as pl
from jax.experimental.pallas import tpu as pltpu


IMGS = 1

_G44 = (44, 44, 48, 2216, 49, 2112)
_G22 = (22, 22, 24, 584, 25, 528)
_G11 = (11, 11, 13, 176, 14, 143)


def _mp(geom):
    return (IMGS - 1) * geom[3] + geom[5]


def _taps(src, base, Wp, MP, Cin, KH, KW, w_ref, b_ref, relu, im=None):
    def slab(kh, kw):
        return src[pl.ds(base + kh * Wp + kw, MP), pl.ds(0, Cin)]

    if im is not None:
        T = KH * KW
        for kh in range(KH):
            for kw in range(KW):
                im[pl.ds(0, MP), pl.ds((kh * KW + kw) * Cin, Cin)] = (
                    slab(kh, kw))
        res = jnp.dot(im[pl.ds(0, MP), pl.ds(0, T * Cin)], w_ref[...],
                      preferred_element_type=jnp.float32) + b_ref[...]
    else:
        res = None
        for kh in range(KH):
            for kw in range(KW):
                d = jnp.dot(slab(kh, kw), w_ref[kh, kw],
                            preferred_element_type=jnp.float32)
                res = (d + b_ref[...]) if res is None else (res + d)
    if relu:
        res = jnp.maximum(res, 0.0)
    return res


def _store(dst, val, geom, lane_off, C):
    PAD, off, M = geom[3], geom[4], geom[5]
    for i in range(IMGS):
        dst[pl.ds(i * PAD + off, M),
            pl.ds(lane_off, C)] = val[i * PAD:i * PAD + M].astype(
                jnp.bfloat16)


def _halo(dst, geom, C):
    H, W, Wp, PAD, off, M = geom
    side = jnp.zeros((Wp - W, C), jnp.bfloat16)
    for i in range(IMGS):
        b = i * PAD
        dst[pl.ds(b, Wp + 1), pl.ds(0, C)] = jnp.zeros((Wp + 1, C),
                                                       jnp.bfloat16)
        for r in range(1, H + 1):
            dst[pl.ds(b + r * Wp + W + 1, Wp - W), pl.ds(0, C)] = side
        t0 = (H + 1) * Wp + 1
        dst[pl.ds(b + t0, PAD - t0), pl.ds(0, C)] = jnp.zeros(
            (PAD - t0, C), jnp.bfloat16)


def _conv_to(src, sgeom, dst, dgeom, Cin, w_ref, b_ref, relu,
             lane_off=0, O=None, im=None):
    res = _taps(src, 0, sgeom[2], _mp(sgeom), Cin, 3, 3, w_ref, b_ref,
                relu, im=im)
    _store(dst, res, dgeom, lane_off, O)


def _max_pool_2x2(src, sgeom, dst, dgeom, C):
    H, W, Wp, sPAD = sgeom[0], sgeom[1], sgeom[2], sgeom[3]
    OH, OW = H // 2, W // 2
    ri = jax.lax.broadcasted_iota(jnp.int32, (OW, W), 0)
    cj = jax.lax.broadcasted_iota(jnp.int32, (OW, W), 1)
    sel_e = jnp.where(cj == 2 * ri, 1.0, 0.0).astype(jnp.bfloat16)
    sel_o = jnp.where(cj == 2 * ri + 1, 1.0, 0.0).astype(jnp.bfloat16)
    dWp, dPAD = dgeom[2], dgeom[3]
    for i in range(IMGS):
        for oh in range(OH):
            r0 = src[pl.ds(i * sPAD + (2 * oh + 1) * Wp + 1, W),
                     pl.ds(0, C)]
            r1 = src[pl.ds(i * sPAD + (2 * oh + 2) * Wp + 1, W),
                     pl.ds(0, C)]
            rm = jnp.maximum(r0, r1)
            e = jnp.dot(sel_e, rm, preferred_element_type=jnp.float32)
            o = jnp.dot(sel_o, rm, preferred_element_type=jnp.float32)
            dst[pl.ds(i * dPAD + (oh + 1) * dWp + 1, OW),
                pl.ds(0, C)] = jnp.maximum(e, o).astype(jnp.bfloat16)


def _max_pool_3x3(src, dst, geom, C):
    Wp, PAD, off, M = geom[2], geom[3], geom[4], geom[5]
    MP = _mp(geom)
    m = None
    for i in range(3):
        for j in range(3):
            sl = src[pl.ds(i * Wp + j, MP), pl.ds(0, C)]
            m = sl if m is None else jnp.maximum(m, sl)
    for i in range(IMGS):
        dst[pl.ds(i * PAD + off, M), pl.ds(0, C)] = m[i * PAD:i * PAD + M]


def _feat(src, sgeom, w_ref, b_ref, dst, fpc, Cin, im=None):
    H, W, Wp, PAD, off, M = sgeom
    MP = _mp(sgeom)
    f = _taps(src, 0, Wp, MP, Cin, 3, 3, w_ref, b_ref, True, im=im)
    _store(dst, f, sgeom, 0, 64)
    fpc[pl.ds(0, IMGS * PAD), pl.ds(0, 64)] = jnp.zeros(
        (IMGS * PAD, 64), jnp.float32)
    for i in range(IMGS):
        for oh in range(H):
            fpc[pl.ds(i * PAD + (oh + 1) * Wp + 1, W), pl.ds(0, 64)] = (
                f[i * PAD + oh * Wp:i * PAD + oh * Wp + W])
    s = None
    for i in range(3):
        for j in range(3):
            sl = fpc[pl.ds(i * Wp + j, MP), pl.ds(0, 64)]
            s = sl if s is None else s + sl
    _store(dst, f - s * (1.0 / 9.0), sgeom, 64, 64)


def kernel(x, base0_w, base0_b, base1_w, base1_b, base2_w, base2_b,
           base3_w, base3_b, base4_w, base4_b, base5_w, base5_b,
           base6_w, base6_b, base7_w, base7_b, base8_w, base8_b,
           base9_w, base9_b, base10_w, base10_b, base11_w, base11_b,
           base12_w, base12_b,
           feat0_w, feat0_b, feat1_w, feat1_b, feat2_w, feat2_b,
           feat3_w, feat3_b, feat4_w, feat4_b,
           pool0_w, pool0_b, pool1_w, pool1_b, pool2_w, pool2_b,
           pool3_w, pool3_b, pool4_w, pool4_b,
           glob0_w, glob0_b, glob1_w, glob1_b, glob2_w, glob2_b,
           conv_g_w, conv_g_b, conv_l_w, conv_l_b):
    N = x.shape[0]
    G = N // IMGS
    x_nhwc = jnp.transpose(x.astype(jnp.float32), (0, 2, 3, 1))
    x_pad = jnp.pad(x_nhwc, ((0, 0), (1, 1), (1, 3), (0, 0)))
    x_pad = x_pad.reshape(N, 46 * 48, 3)
    x_pad = jnp.pad(x_pad, ((0, 0), (0, 106), (0, 0))).astype(jnp.bfloat16)
    cols = []
    for kh in range(3):
        for kw in range(3):
            d = kh * 48 + kw
            cols.append(jax.lax.dynamic_slice_in_dim(x_pad, d, 2216, axis=1))
    x_im = jnp.concatenate(cols, axis=-1)
    x_im = x_im.reshape(G, IMGS * 2216, 27)

    base_w = [base0_w, base1_w, base2_w, base3_w, base4_w, base5_w, base6_w,
              base7_w, base8_w, base9_w, base10_w, base11_w, base12_w]
    base_b = [base0_b, base1_b, base2_b, base3_b, base4_b, base5_b, base6_b,
              base7_b, base8_b, base9_b, base10_b, base11_b, base12_b]
    feat_w = [feat0_w, feat1_w, feat2_w, feat3_w, feat4_w]
    feat_b = [feat0_b, feat1_b, feat2_b, feat3_b, feat4_b]
    pool_w = [pool0_w, pool1_w, pool2_w, pool3_w, pool4_w]
    pool_b = [pool0_b, pool1_b, pool2_b, pool3_b, pool4_b]
    glob_w = [glob0_w, glob1_w, glob2_w]
    glob_b = [glob0_b, glob1_b, glob2_b]

    operands = [x_im]
    for w, b in zip(base_w + feat_w + pool_w + glob_w,
                    base_b + feat_b + pool_b + glob_b):
        KH, KW, Cin, O = w.shape
        wb = w.astype(jnp.bfloat16)
        if Cin <= 128:
            wb = wb.reshape(KH * KW * Cin, O)
        operands.append(wb)
        operands.append(b.reshape(1, -1).astype(jnp.float32))
    operands.append(conv_l_w.reshape(1, 640).astype(jnp.float32))
    operands.append(conv_g_w.reshape(1, 128).astype(jnp.float32))
    operands.append((conv_l_b + conv_g_b).reshape(1, 1).astype(jnp.float32))

    in_specs = [pl.BlockSpec((1, IMGS * 2216, 27), lambda n: (n, 0, 0))]
    for op in operands[1:]:
        in_specs.append(
            pl.BlockSpec(op.shape, lambda n, nd=op.ndim: (0,) * nd))

    def body(*refs):
        x_ref = refs[0]
        wr = refs[1:53]
        wl_ref, wg_ref, sb_ref = refs[53], refs[54], refs[55]
        prob_ref = refs[56]
        (c44a, c44b, c22a, c22b, c11a, c11b,
         dc0, dc1, dc2, dc3, dc4, gb1, gb2, fpc,
         im44, imm, img) = refs[57:]

        bw = [(wr[2 * i], wr[2 * i + 1]) for i in range(13)]
        fw = [(wr[2 * i + 26], wr[2 * i + 27]) for i in range(5)]
        pw = [(wr[2 * i + 36], wr[2 * i + 37]) for i in range(5)]
        gw = [(wr[2 * i + 46], wr[2 * i + 47]) for i in range(3)]

        res0 = jnp.dot(x_ref[0, pl.ds(0, _mp(_G44)), :], bw[0][0][...],
                       preferred_element_type=jnp.float32) + bw[0][1][...]
        _store(c44a, jnp.maximum(res0, 0.0), _G44, 0, 16)
        _halo(c44a, _G44, 16)
        _conv_to(c44a, _G44, c44b, _G44, 16, *bw[1], True, O=16, im=im44)
        _halo(c44b, _G44, 16)
        _max_pool_2x2(c44b, _G44, c22a, _G22, 16)
        _halo(c22a, _G22, 16)
        _feat(c22a, _G22, *fw[0], dc0, fpc, 16, im=imm)
        _conv_to(c22a, _G22, c22b, _G22, 16, *bw[2], True, O=32, im=imm)
        _halo(c22b, _G22, 32)
        _conv_to(c22b, _G22, c22a, _G22, 32, *bw[3], True, O=32, im=imm)
        _halo(c22a, _G22, 32)
        _max_pool_2x2(c22a, _G22, c11a, _G11, 32)
        _halo(c11a, _G11, 32)
        _feat(c11a, _G11, *fw[1], dc1, fpc, 32, im=imm)
        _conv_to(c11a, _G11, c11b, _G11, 32, *bw[4], True, O=32, im=imm)
        _halo(c11b, _G11, 32)
        _conv_to(c11b, _G11, c11a, _G11, 32, *bw[5], True, O=32, im=imm)
        _halo(c11a, _G11, 32)
        _conv_to(c11a, _G11, c11b, _G11, 32, *bw[6], True, O=32, im=imm)
        _halo(c11b, _G11, 32)
        _max_pool_3x3(c11b, c11a, _G11, 32)
        _halo(c11a, _G11, 32)
        _feat(c11a, _G11, *fw[2], dc2, fpc, 32, im=imm)
        _conv_to(c11a, _G11, c11b, _G11, 32, *bw[7], True, O=64, im=imm)
        _halo(c11b, _G11, 64)
        _conv_to(c11b, _G11, c11a, _G11, 64, *bw[8], True, O=64, im=imm)
        _halo(c11a, _G11, 64)
        _conv_to(c11a, _G11, c11b, _G11, 64, *bw[9], True, O=64, im=imm)
        _halo(c11b, _G11, 64)
        _max_pool_3x3(c11b, c11a, _G11, 64)
        _halo(c11a, _G11, 64)
        _feat(c11a, _G11, *fw[3], dc3, fpc, 64, im=imm)
        _conv_to(c11a, _G11, c11b, _G11, 64, *bw[10], True, O=512, im=imm)
        _halo(c11b, _G11, 512)
        _conv_to(c11b, _G11, c11a, _G11, 512, *bw[11], True, O=512)
        _halo(c11a, _G11, 512)
        _conv_to(c11a, _G11, c11b, _G11, 512, *bw[12], True, O=512)
        _halo(c11b, _G11, 512)
        _max_pool_3x3(c11b, c11a, _G11, 512)
        _halo(c11a, _G11, 512)
        _feat(c11a, _G11, *fw[4], dc4, fpc, 512)

        g0 = _taps(c11a, 14, 13, (IMGS - 1) * 176 + 85, 512, 5, 5,
                   *gw[0], True)
        for i in range(IMGS):
            gb1[pl.ds(i * 88, 85), :] = g0[i * 176:i * 176 + 85].astype(
                jnp.bfloat16)
        g1 = _taps(gb1, 0, 13, (IMGS - 1) * 88 + 29, 128, 5, 5,
                   *gw[1], True, im=img)
        for i in range(IMGS):
            gb2[pl.ds(i * 32, 29), :] = g1[i * 88:i * 88 + 29].astype(
                jnp.bfloat16)
        g2 = _taps(gb2, 0, 13, (IMGS - 1) * 32 + 1, 128, 3, 3,
                   *gw[2], False, im=imm)
        gsv = jnp.sum(g2 * wg_ref[...], axis=-1, keepdims=True)

        _halo(dc4, _G11, 128)
        d = _taps(dc4, 0, 13, _mp(_G11), 128, 3, 3, *pw[4], True, im=imm)
        for k, dck in ((3, dc3), (2, dc2), (1, dc1)):
            _store(dck, d, _G11, 128, 128)
            _halo(dck, _G11, 256)
            d = _taps(dck, 0, 13, _mp(_G11), 256, 3, 3, *pw[k], True)
        ri = jax.lax.broadcasted_iota(jnp.int32, (22, 11), 0)
        cj = jax.lax.broadcasted_iota(jnp.int32, (22, 11), 1)
        sel = jnp.where(ri // 2 == cj, 1.0, 0.0)
        for i in range(IMGS):
            for hs in range(11):
                b0 = i * 176 + hs * 13
                row = jnp.dot(sel, d[b0:b0 + 11],
                              preferred_element_type=jnp.float32)
                row = row.astype(jnp.bfloat16)
                for r in range(2):
                    dc0[pl.ds(i * 584 + (2 * hs + r + 1) * 24 + 1, 22),
                        pl.ds(128, 128)] = row
        _halo(dc0, _G22, 256)
        out = _taps(dc0, 0, 24, _mp(_G22), 256, 3, 3, *pw[0], False)

        s = jnp.sum(out * wl_ref[...], axis=-1, keepdims=True)
        for i in range(IMGS):
            z = (s[i * 584:i * 584 + 528] + gsv[i * 32:i * 32 + 1]
                 + sb_ref[...])
            prob_ref[i] = 1.0 / (1.0 + jnp.exp(-z))

    scratch = [
        pltpu.VMEM((IMGS * 2216, 16), jnp.bfloat16),
        pltpu.VMEM((IMGS * 2216, 16), jnp.bfloat16),
        pltpu.VMEM((IMGS * 584, 32), jnp.bfloat16),
        pltpu.VMEM((IMGS * 584, 32), jnp.bfloat16),
        pltpu.VMEM((IMGS * 176, 512), jnp.bfloat16),
        pltpu.VMEM((IMGS * 176, 512), jnp.bfloat16),
        pltpu.VMEM((IMGS * 584, 256), jnp.bfloat16),
        pltpu.VMEM((IMGS * 176, 256), jnp.bfloat16),
        pltpu.VMEM((IMGS * 176, 256), jnp.bfloat16),
        pltpu.VMEM((IMGS * 176, 256), jnp.bfloat16),
        pltpu.VMEM((IMGS * 176, 256), jnp.bfloat16),
        pltpu.VMEM((IMGS * 88, 128), jnp.bfloat16),
        pltpu.VMEM((IMGS * 32, 128), jnp.bfloat16),
        pltpu.VMEM((IMGS * 584, 64), jnp.float32),
        pltpu.VMEM(((IMGS - 1) * 2216 + 2112, 144), jnp.bfloat16),
        pltpu.VMEM(((IMGS - 1) * 584 + 528, 1152), jnp.bfloat16),
        pltpu.VMEM(((IMGS - 1) * 88 + 32, 3200), jnp.bfloat16),
    ]

    prob = pl.pallas_call(
        body,
        out_shape=jax.ShapeDtypeStruct((N, 528, 1), jnp.float32),
        grid=(G,),
        in_specs=in_specs,
        out_specs=pl.BlockSpec((IMGS, 528, 1), lambda n: (n, 0, 0)),
        scratch_shapes=scratch,
        compiler_params=pltpu.CompilerParams(
            dimension_semantics=("parallel",),
            vmem_limit_bytes=100 * 1024 * 1024),
    )(*operands)

    prob = prob.reshape(N, 22, 24)[:, :, :22]
    return prob[:, None, :, :]


# speedup vs baseline: 1.1818x; 1.0218x over previous
import jax
import jax.numpy as jnp
from jax.experimental import pallas as pl
from jax.experimental.pallas import tpu as pltpu


IMGS = 1

_G44 = (44, 44, 48, 2224, 56, 2112)
_G22 = (22, 22, 24, 592, 32, 528)
_G11 = (11, 11, 13, 176, 16, 143)


def _mp(geom):
    return (IMGS - 1) * geom[3] + geom[5]


def _taps(src, base, Wp, MP, Cin, KH, KW, w_ref, b_ref, relu, im=None):
    def slab(kh, kw):
        return src[pl.ds(base + kh * Wp + kw, MP), pl.ds(0, Cin)]

    if im is not None:
        T = KH * KW
        for kh in range(KH):
            for kw in range(KW):
                im[pl.ds(0, MP), pl.ds((kh * KW + kw) * Cin, Cin)] = (
                    slab(kh, kw))
        res = jnp.dot(im[pl.ds(0, MP), pl.ds(0, T * Cin)], w_ref[...],
                      preferred_element_type=jnp.float32) + b_ref[...]
    else:
        res = None
        for kh in range(KH):
            for kw in range(KW):
                d = jnp.dot(slab(kh, kw), w_ref[kh, kw],
                            preferred_element_type=jnp.float32)
                res = (d + b_ref[...]) if res is None else (res + d)
    if relu:
        res = jnp.maximum(res, 0.0)
    return res


def _base(geom):
    return geom[4] - geom[2] - 1


def _store(dst, val, geom, lane_off, C):
    PAD, off, M = geom[3], geom[4], geom[5]
    for i in range(IMGS):
        dst[pl.ds(i * PAD + off, M),
            pl.ds(lane_off, C)] = val[i * PAD:i * PAD + M].astype(
                jnp.bfloat16)


def _halo(dst, geom, C):
    H, W, Wp, PAD, off, M = geom
    side = jnp.zeros((Wp - W, C), jnp.bfloat16)
    for i in range(IMGS):
        b = i * PAD
        dst[pl.ds(b, off), pl.ds(0, C)] = jnp.zeros((off, C), jnp.bfloat16)
        for h in range(H):
            dst[pl.ds(b + off + h * Wp + W, Wp - W), pl.ds(0, C)] = side
        t0 = off + H * Wp
        dst[pl.ds(b + t0, PAD - t0), pl.ds(0, C)] = jnp.zeros(
            (PAD - t0, C), jnp.bfloat16)


def _conv_to(src, sgeom, dst, dgeom, Cin, w_ref, b_ref, relu,
             lane_off=0, O=None, im=None):
    res = _taps(src, _base(sgeom), sgeom[2], _mp(sgeom), Cin, 3, 3,
                w_ref, b_ref, relu, im=im)
    _store(dst, res, dgeom, lane_off, O)


def _max_pool_2x2(src, sgeom, dst, dgeom, C):
    H, W, Wp, sPAD = sgeom[0], sgeom[1], sgeom[2], sgeom[3]
    OH, OW = H // 2, W // 2
    ri = jax.lax.broadcasted_iota(jnp.int32, (OW, W), 0)
    cj = jax.lax.broadcasted_iota(jnp.int32, (OW, W), 1)
    sel_e = jnp.where(cj == 2 * ri, 1.0, 0.0).astype(jnp.bfloat16)
    sel_o = jnp.where(cj == 2 * ri + 1, 1.0, 0.0).astype(jnp.bfloat16)
    s_off = sgeom[4]
    dWp, dPAD, d_off = dgeom[2], dgeom[3], dgeom[4]
    for i in range(IMGS):
        for oh in range(OH):
            r0 = src[pl.ds(i * sPAD + s_off + 2 * oh * Wp, W),
                     pl.ds(0, C)]
            r1 = src[pl.ds(i * sPAD + s_off + (2 * oh + 1) * Wp, W),
                     pl.ds(0, C)]
            rm = jnp.maximum(r0, r1)
            e = jnp.dot(sel_e, rm, preferred_element_type=jnp.float32)
            o = jnp.dot(sel_o, rm, preferred_element_type=jnp.float32)
            dst[pl.ds(i * dPAD + d_off + oh * dWp, OW),
                pl.ds(0, C)] = jnp.maximum(e, o).astype(jnp.bfloat16)


def _max_pool_3x3(src, dst, geom, C):
    Wp, PAD, off, M = geom[2], geom[3], geom[4], geom[5]
    MP = _mp(geom)
    m = None
    for i in range(3):
        for j in range(3):
            sl = src[pl.ds(_base(geom) + i * Wp + j, MP), pl.ds(0, C)]
            m = sl if m is None else jnp.maximum(m, sl)
    for i in range(IMGS):
        dst[pl.ds(i * PAD + off, M), pl.ds(0, C)] = m[i * PAD:i * PAD + M]


def _feat(src, sgeom, w_ref, b_ref, dst, fpc, Cin, im=None):
    H, W, Wp, PAD, off, M = sgeom
    MP = _mp(sgeom)
    f = _taps(src, _base(sgeom), Wp, MP, Cin, 3, 3, w_ref, b_ref, True,
              im=im)
    _store(dst, f, sgeom, 0, 64)
    fpc[pl.ds(0, IMGS * PAD), pl.ds(0, 64)] = jnp.zeros(
        (IMGS * PAD, 64), jnp.float32)
    for i in range(IMGS):
        for oh in range(H):
            fpc[pl.ds(i * PAD + off + oh * Wp, W), pl.ds(0, 64)] = (
                f[i * PAD + oh * Wp:i * PAD + oh * Wp + W])
    s = None
    for i in range(3):
        for j in range(3):
            sl = fpc[pl.ds(_base(sgeom) + i * Wp + j, MP), pl.ds(0, 64)]
            s = sl if s is None else s + sl
    _store(dst, f - s * (1.0 / 9.0), sgeom, 64, 64)


def kernel(x, base0_w, base0_b, base1_w, base1_b, base2_w, base2_b,
           base3_w, base3_b, base4_w, base4_b, base5_w, base5_b,
           base6_w, base6_b, base7_w, base7_b, base8_w, base8_b,
           base9_w, base9_b, base10_w, base10_b, base11_w, base11_b,
           base12_w, base12_b,
           feat0_w, feat0_b, feat1_w, feat1_b, feat2_w, feat2_b,
           feat3_w, feat3_b, feat4_w, feat4_b,
           pool0_w, pool0_b, pool1_w, pool1_b, pool2_w, pool2_b,
           pool3_w, pool3_b, pool4_w, pool4_b,
           glob0_w, glob0_b, glob1_w, glob1_b, glob2_w, glob2_b,
           conv_g_w, conv_g_b, conv_l_w, conv_l_b):
    N = x.shape[0]
    G = N // IMGS
    x_nhwc = jnp.transpose(x.astype(jnp.float32), (0, 2, 3, 1))
    x_pad = jnp.pad(x_nhwc, ((0, 0), (1, 1), (1, 3), (0, 0)))
    x_pad = x_pad.reshape(N, 46 * 48, 3)
    x_pad = jnp.pad(x_pad, ((0, 0), (0, 106), (0, 0))).astype(jnp.bfloat16)
    cols = []
    for kh in range(3):
        for kw in range(3):
            d = kh * 48 + kw
            cols.append(jax.lax.dynamic_slice_in_dim(x_pad, d, 2216, axis=1))
    x_im = jnp.concatenate(cols, axis=-1)
    x_im = x_im.reshape(G, IMGS * 2216, 27)

    base_w = [base0_w, base1_w, base2_w, base3_w, base4_w, base5_w, base6_w,
              base7_w, base8_w, base9_w, base10_w, base11_w, base12_w]
    base_b = [base0_b, base1_b, base2_b, base3_b, base4_b, base5_b, base6_b,
              base7_b, base8_b, base9_b, base10_b, base11_b, base12_b]
    feat_w = [feat0_w, feat1_w, feat2_w, feat3_w, feat4_w]
    feat_b = [feat0_b, feat1_b, feat2_b, feat3_b, feat4_b]
    pool_w = [pool0_w, pool1_w, pool2_w, pool3_w, pool4_w]
    pool_b = [pool0_b, pool1_b, pool2_b, pool3_b, pool4_b]
    glob_w = [glob0_w, glob1_w, glob2_w]
    glob_b = [glob0_b, glob1_b, glob2_b]

    operands = [x_im]
    for w, b in zip(base_w + feat_w + pool_w + glob_w,
                    base_b + feat_b + pool_b + glob_b):
        KH, KW, Cin, O = w.shape
        wb = w.astype(jnp.bfloat16)
        if Cin <= 128:
            wb = wb.reshape(KH * KW * Cin, O)
        operands.append(wb)
        operands.append(b.reshape(1, -1).astype(jnp.float32))
    operands.append(conv_l_w.reshape(1, 640).astype(jnp.float32))
    operands.append(conv_g_w.reshape(1, 128).astype(jnp.float32))
    operands.append((conv_l_b + conv_g_b).reshape(1, 1).astype(jnp.float32))

    in_specs = [pl.BlockSpec((1, IMGS * 2216, 27), lambda n: (n, 0, 0))]
    for op in operands[1:]:
        in_specs.append(
            pl.BlockSpec(op.shape, lambda n, nd=op.ndim: (0,) * nd))

    def body(*refs):
        x_ref = refs[0]
        wr = refs[1:53]
        wl_ref, wg_ref, sb_ref = refs[53], refs[54], refs[55]
        prob_ref = refs[56]
        (c44a, c44b, c22a, c22b, c11a, c11b,
         dc0, dc1, dc2, dc3, dc4, gb1, gb2, fpc,
         im44, imm, img) = refs[57:]

        bw = [(wr[2 * i], wr[2 * i + 1]) for i in range(13)]
        fw = [(wr[2 * i + 26], wr[2 * i + 27]) for i in range(5)]
        pw = [(wr[2 * i + 36], wr[2 * i + 37]) for i in range(5)]
        gw = [(wr[2 * i + 46], wr[2 * i + 47]) for i in range(3)]

        res0 = jnp.dot(x_ref[0, pl.ds(0, _mp(_G44)), :], bw[0][0][...],
                       preferred_element_type=jnp.float32) + bw[0][1][...]
        _store(c44a, jnp.maximum(res0, 0.0), _G44, 0, 16)
        _halo(c44a, _G44, 16)
        _conv_to(c44a, _G44, c44b, _G44, 16, *bw[1], True, O=16, im=im44)
        _halo(c44b, _G44, 16)
        _max_pool_2x2(c44b, _G44, c22a, _G22, 16)
        _halo(c22a, _G22, 16)
        _feat(c22a, _G22, *fw[0], dc0, fpc, 16, im=imm)
        _conv_to(c22a, _G22, c22b, _G22, 16, *bw[2], True, O=32, im=imm)
        _halo(c22b, _G22, 32)
        _conv_to(c22b, _G22, c22a, _G22, 32, *bw[3], True, O=32, im=imm)
        _halo(c22a, _G22, 32)
        _max_pool_2x2(c22a, _G22, c11a, _G11, 32)
        _halo(c11a, _G11, 32)
        _feat(c11a, _G11, *fw[1], dc1, fpc, 32, im=imm)
        _conv_to(c11a, _G11, c11b, _G11, 32, *bw[4], True, O=32, im=imm)
        _halo(c11b, _G11, 32)
        _conv_to(c11b, _G11, c11a, _G11, 32, *bw[5], True, O=32, im=imm)
        _halo(c11a, _G11, 32)
        _conv_to(c11a, _G11, c11b, _G11, 32, *bw[6], True, O=32, im=imm)
        _halo(c11b, _G11, 32)
        _max_pool_3x3(c11b, c11a, _G11, 32)
        _halo(c11a, _G11, 32)
        _feat(c11a, _G11, *fw[2], dc2, fpc, 32, im=imm)
        _conv_to(c11a, _G11, c11b, _G11, 32, *bw[7], True, O=64, im=imm)
        _halo(c11b, _G11, 64)
        _conv_to(c11b, _G11, c11a, _G11, 64, *bw[8], True, O=64, im=imm)
        _halo(c11a, _G11, 64)
        _conv_to(c11a, _G11, c11b, _G11, 64, *bw[9], True, O=64, im=imm)
        _halo(c11b, _G11, 64)
        _max_pool_3x3(c11b, c11a, _G11, 64)
        _halo(c11a, _G11, 64)
        _feat(c11a, _G11, *fw[3], dc3, fpc, 64, im=imm)
        _conv_to(c11a, _G11, c11b, _G11, 64, *bw[10], True, O=512, im=imm)
        _halo(c11b, _G11, 512)
        _conv_to(c11b, _G11, c11a, _G11, 512, *bw[11], True, O=512)
        _halo(c11a, _G11, 512)
        _conv_to(c11a, _G11, c11b, _G11, 512, *bw[12], True, O=512)
        _halo(c11b, _G11, 512)
        _max_pool_3x3(c11b, c11a, _G11, 512)
        _halo(c11a, _G11, 512)
        _feat(c11a, _G11, *fw[4], dc4, fpc, 512)

        g0 = _taps(c11a, 16, 13, (IMGS - 1) * 176 + 85, 512, 5, 5,
                   *gw[0], True)
        for i in range(IMGS):
            gb1[pl.ds(i * 88, 85), :] = g0[i * 176:i * 176 + 85].astype(
                jnp.bfloat16)
        g1 = _taps(gb1, 0, 13, (IMGS - 1) * 88 + 29, 128, 5, 5,
                   *gw[1], True, im=img)
        for i in range(IMGS):
            gb2[pl.ds(i * 32, 29), :] = g1[i * 88:i * 88 + 29].astype(
                jnp.bfloat16)
        g2 = _taps(gb2, 0, 13, (IMGS - 1) * 32 + 1, 128, 3, 3,
                   *gw[2], False, im=imm)
        gsv = jnp.sum(g2 * wg_ref[...], axis=-1, keepdims=True)

        _halo(dc4, _G11, 128)
        d = _taps(dc4, _base(_G11), 13, _mp(_G11), 128, 3, 3, *pw[4],
                  True, im=imm)
        for k, dck in ((3, dc3), (2, dc2), (1, dc1)):
            _store(dck, d, _G11, 128, 128)
            _halo(dck, _G11, 256)
            d = _taps(dck, _base(_G11), 13, _mp(_G11), 256, 3, 3, *pw[k],
                      True)
        ri = jax.lax.broadcasted_iota(jnp.int32, (22, 11), 0)
        cj = jax.lax.broadcasted_iota(jnp.int32, (22, 11), 1)
        sel = jnp.where(ri // 2 == cj, 1.0, 0.0)
        for i in range(IMGS):
            for hs in range(11):
                b0 = i * 176 + hs * 13
                row = jnp.dot(sel, d[b0:b0 + 11],
                              preferred_element_type=jnp.float32)
                row = row.astype(jnp.bfloat16)
                for r in range(2):
                    dc0[pl.ds(i * 592 + 32 + (2 * hs + r) * 24, 22),
                        pl.ds(128, 128)] = row
        _halo(dc0, _G22, 256)
        out = _taps(dc0, _base(_G22), 24, _mp(_G22), 256, 3, 3, *pw[0],
                    False)

        s = jnp.sum(out * wl_ref[...], axis=-1, keepdims=True)
        for i in range(IMGS):
            z = (s[i * 592:i * 592 + 528] + gsv[i * 32:i * 32 + 1]
                 + sb_ref[...])
            prob_ref[i] = 1.0 / (1.0 + jnp.exp(-z))

    scratch = [
        pltpu.VMEM((IMGS * 2224, 16), jnp.bfloat16),
        pltpu.VMEM((IMGS * 2224, 16), jnp.bfloat16),
        pltpu.VMEM((IMGS * 592, 32), jnp.bfloat16),
        pltpu.VMEM((IMGS * 592, 32), jnp.bfloat16),
        pltpu.VMEM((IMGS * 176, 512), jnp.bfloat16),
        pltpu.VMEM((IMGS * 176, 512), jnp.bfloat16),
        pltpu.VMEM((IMGS * 592, 256), jnp.bfloat16),
        pltpu.VMEM((IMGS * 176, 256), jnp.bfloat16),
        pltpu.VMEM((IMGS * 176, 256), jnp.bfloat16),
        pltpu.VMEM((IMGS * 176, 256), jnp.bfloat16),
        pltpu.VMEM((IMGS * 176, 256), jnp.bfloat16),
        pltpu.VMEM((IMGS * 88, 128), jnp.bfloat16),
        pltpu.VMEM((IMGS * 32, 128), jnp.bfloat16),
        pltpu.VMEM((IMGS * 592, 64), jnp.float32),
        pltpu.VMEM(((IMGS - 1) * 2224 + 2112, 144), jnp.bfloat16),
        pltpu.VMEM(((IMGS - 1) * 592 + 528, 1152), jnp.bfloat16),
        pltpu.VMEM(((IMGS - 1) * 88 + 32, 3200), jnp.bfloat16),
    ]

    prob = pl.pallas_call(
        body,
        out_shape=jax.ShapeDtypeStruct((N, 528, 1), jnp.float32),
        grid=(G,),
        in_specs=in_specs,
        out_specs=pl.BlockSpec((IMGS, 528, 1), lambda n: (n, 0, 0)),
        scratch_shapes=scratch,
        compiler_params=pltpu.CompilerParams(
            dimension_semantics=("parallel",),
            vmem_limit_bytes=100 * 1024 * 1024),
    )(*operands)

    prob = prob.reshape(N, 22, 24)[:, :, :22]
    return prob[:, None, :, :]
